# baseline (speedup 1.0000x reference)
import functools

import jax
import jax.numpy as jnp
from jax import lax
from jax.experimental import pallas as pl
from jax.experimental.pallas import tpu as pltpu
from jax.experimental.pallas import tpu_sc as plsc

NUM_FIELDS = 26
EMBED_DIM = 32
BATCH = 16384
N = BATCH * NUM_FIELDS
NC, NS, L = 2, 16, 16
NW = NC * NS
PER_W = N // NW
ROWS = PER_W // 128
G = 13
GROUPS = ROWS // G
CHUNK = G * 128


def _sc_gather(idx3, pat, table):
    mesh = plsc.VectorSubcoreMesh(
        core_axis_name="c", subcore_axis_name="s", num_cores=NC, num_subcores=NS
    )

    @functools.partial(
        pl.kernel,
        mesh=mesh,
        compiler_params=pltpu.CompilerParams(use_tc_tiling_on_sc=False),
        out_type=jax.ShapeDtypeStruct((N, EMBED_DIM), jnp.float32),
        scratch_types=[
            pltpu.VMEM((ROWS, 128), jnp.int32),
            pltpu.VMEM((ROWS, 128), jnp.int32),
            pltpu.VMEM((CHUNK, EMBED_DIM), jnp.float32),
            pltpu.SemaphoreType.DMA,
        ],
    )
    def k(idx_hbm, pat_hbm, table_hbm, out_hbm, idx_v, pat_v, rows_v, sem):
        wid = lax.axis_index("s") * NC + lax.axis_index("c")
        pltpu.sync_copy(idx_hbm.at[wid], idx_v)
        pltpu.sync_copy(pat_hbm, pat_v)

        def add_row(g, c):
            for j in range(128 // L):
                sl = pl.ds(j * L, L)
                idx_v[g, sl] = idx_v[g, sl] + pat_v[g, sl]
            return c

        lax.fori_loop(0, ROWS, add_row, 0)

        def do_group(gr, c):
            cps = [
                pltpu.async_copy(
                    table_hbm.at[idx_v.at[gr * G + t]],
                    rows_v.at[pl.ds(t * 128, 128)],
                    sem,
                )
                for t in range(G)
            ]
            for cp in cps:
                cp.wait()
            base = wid * PER_W + gr * CHUNK
            pltpu.sync_copy(rows_v, out_hbm.at[pl.ds(base, CHUNK)])
            return c

        lax.fori_loop(0, GROUPS, do_group, 0)

    return k(idx3, pat, table)


def kernel(input, num_classes, table):
    offsets = jnp.concatenate(
        [jnp.zeros((1,), dtype=num_classes.dtype), jnp.cumsum(num_classes)[:-1]]
    ).astype(jnp.int32)
    pat = jnp.tile(offsets, PER_W // NUM_FIELDS).reshape(ROWS, 128)
    idx3 = input.reshape(NW, ROWS, 128)
    out = _sc_gather(idx3, pat, table)
    return out.reshape(BATCH, NUM_FIELDS * EMBED_DIM)
